# baseline (speedup 1.0000x reference)
import functools

import numpy as np

import jax
import jax.numpy as jnp
from jax import lax
from jax.experimental import pallas as pl
from jax.experimental.pallas import tpu as pltpu
from jax.experimental.pallas import tpu_sc as plsc

B, N, S, NS = 8, 4096, 1024, 8
D1, D2 = 128, 256
C1 = 128
C2 = 64
C3 = 64
SPLIT = 4
BH = B // SPLIT
TN = 512
TM = 512
TB = 1024
CNT1 = float(B * N * NS)
CNT3 = float(B * N)
NV = C1 // 16

_j = np.arange(C1 // 2)
CA = (32 * (_j // 16) + (_j % 16)).astype(np.int32)
CB = CA + 16
CACB = np.concatenate([CA, CB])


def _table_body(feat2_ref, xyz2_ref, wft_ref, wxt_ref, b_ref, gg_ref):
    gg = jnp.dot(feat2_ref[0], wft_ref[...], preferred_element_type=jnp.float32)
    gg += jnp.dot(xyz2_ref[0], wxt_ref[...], preferred_element_type=jnp.float32)
    gg_ref[...] = gg + b_ref[...]


def _make_table(feat2, xyz2, wft, wxt, b1_0r):
    return pl.pallas_call(
        _table_body,
        grid=(BH,),
        in_specs=[
            pl.BlockSpec((1, S, D2), lambda b: (b, 0, 0)),
            pl.BlockSpec((1, S, 3), lambda b: (b, 0, 0)),
            pl.BlockSpec((D2, C1), lambda b: (0, 0)),
            pl.BlockSpec((3, C1), lambda b: (0, 0)),
            pl.BlockSpec((1, C1), lambda b: (0, 0)),
        ],
        out_specs=pl.BlockSpec((S, C1), lambda b: (b, 0)),
        out_shape=jax.ShapeDtypeStruct((BH * S, C1), jnp.float32),
    )(feat2, xyz2, wft, wxt, b1_0r)


def _knn_body(xyz1_ref, xyz2t_ref, wxt_ref, idx_ref, xw1_ref):
    b = pl.program_id(0)
    x1 = xyz1_ref[0]
    x2t = xyz2t_ref[0]
    d = -2.0 * jnp.dot(x1, x2t, preferred_element_type=jnp.float32)
    d += jnp.sum(x1 * x1, axis=1, keepdims=True)
    d += jnp.sum(x2t * x2t, axis=0, keepdims=True)
    iota = lax.broadcasted_iota(jnp.int32, (TN, S), 1)
    off = (b * S).astype(jnp.int32)
    for k in range(NS):
        idxk = jnp.argmin(d, axis=1).astype(jnp.int32)
        d = jnp.where(iota == idxk[:, None], jnp.inf, d)
        idx_ref[0, :, k] = idxk + off
    xw1_ref[...] = jnp.dot(x1, wxt_ref[...], preferred_element_type=jnp.float32)


def _knn(xyz1, xyz2t, wxt):
    return pl.pallas_call(
        _knn_body,
        grid=(BH, N // TN),
        in_specs=[
            pl.BlockSpec((1, TN, 3), lambda b, i: (b, i, 0)),
            pl.BlockSpec((1, 3, S), lambda b, i: (b, 0, 0)),
            pl.BlockSpec((3, C1), lambda b, i: (0, 0)),
        ],
        out_specs=[
            pl.BlockSpec((1, TN, NS), lambda b, i: (b, i, 0)),
            pl.BlockSpec((TN, C1), lambda b, i: (b * (N // TN) + i, 0)),
        ],
        out_shape=[
            jax.ShapeDtypeStruct((BH, N, NS), jnp.int32),
            jax.ShapeDtypeStruct((BH * N, C1), jnp.float32),
        ],
    )(xyz1, xyz2t, wxt)


_NROWS = BH * N * NS
_CHUNK = 128
_NWORK = 32
_RPW = _NROWS // _NWORK
_NCH = _RPW // _CHUNK
_PPW = _RPW // NS
_PPC = _CHUNK // NS


def _sc_gather(gg, idx3, xw1f):
    mesh = plsc.VectorSubcoreMesh(core_axis_name="c", subcore_axis_name="s")

    @functools.partial(
        pl.kernel,
        mesh=mesh,
        out_type=[
            jax.ShapeDtypeStruct((_NROWS, C1 // 2), jnp.int32),
            jax.ShapeDtypeStruct((2 * _NWORK, C1), jnp.float32),
        ],
        scratch_types=[
            pltpu.VMEM((_NCH, _CHUNK), jnp.int32),
            pltpu.VMEM((_PPW, C1), jnp.float32),
            pltpu.VMEM((_CHUNK, C1), jnp.float32),
            pltpu.VMEM((_CHUNK, C1), jnp.float32),
            pltpu.VMEM((_CHUNK, C1 // 2), jnp.int32),
            pltpu.VMEM((_CHUNK, C1 // 2), jnp.int32),
            pltpu.VMEM((2, C1), jnp.float32),
            pltpu.SemaphoreType.DMA,
            pltpu.SemaphoreType.DMA,
        ],
    )
    def k(gg_hbm, idx_hbm, xw_hbm, out_hbm, st_hbm,
          idx_v, xw_v, rows0, rows1, pk0, pk1, acc_v, sem0, sem1):
        wid = lax.axis_index("s") * 2 + lax.axis_index("c")
        base = wid * _RPW
        pltpu.sync_copy(idx_hbm.at[wid], idx_v)
        pltpu.sync_copy(xw_hbm.at[pl.ds(wid * _PPW, _PPW)], xw_v)
        z = jnp.zeros((16,), jnp.float32)
        for v in range(NV):
            acc_v[0, pl.ds(v * 16, 16)] = z
            acc_v[1, pl.ds(v * 16, 16)] = z
        bufs = (rows0, rows1)
        pks = (pk0, pk1)
        sems = (sem0, sem1)
        pltpu.async_copy(gg_hbm.at[idx_v.at[0]], rows0, sem0)
        pltpu.async_copy(gg_hbm.at[idx_v.at[1]], rows1, sem1)

        def pair_body(c2, carry):
            for par in range(2):
                c = c2 * 2 + par
                buf = bufs[par]
                pk = pks[par]
                sem = sems[par]
                pltpu.make_async_copy(
                    gg_hbm.at[pl.ds(0, _CHUNK)], buf, sem).wait()

                def point_body(p, carry2):
                    prow = p * NS
                    for v2 in range(NV // 2):
                        la = pl.ds(v2 * 32, 16)
                        lb = pl.ds(v2 * 32 + 16, 16)
                        lo = pl.ds(v2 * 16, 16)
                        xa = xw_v[c * _PPC + p, la]
                        xb = xw_v[c * _PPC + p, lb]
                        sa = acc_v[0, la]
                        qa = acc_v[1, la]
                        sb = acc_v[0, lb]
                        qb = acc_v[1, lb]
                        for rr in range(NS):
                            ya = buf[prow + rr, la] - xa
                            yb = buf[prow + rr, lb] - xb
                            sa = sa + ya
                            qa = qa + ya * ya
                            sb = sb + yb
                            qb = qb + yb * yb
                            wa = lax.bitcast_convert_type(
                                ya, jnp.int32) + 0x8000
                            wb = lax.bitcast_convert_type(
                                yb, jnp.int32) + 0x8000
                            pk[prow + rr, lo] = (
                                (wb & (-65536)) | ((wa >> 16) & 0xFFFF))
                        acc_v[0, la] = sa
                        acc_v[1, la] = qa
                        acc_v[0, lb] = sb
                        acc_v[1, lb] = qb
                    return carry2

                lax.fori_loop(0, _PPC, point_body, 0)
                pltpu.sync_copy(
                    pk, out_hbm.at[pl.ds(base + c * _CHUNK, _CHUNK)])

                @pl.when(c + 2 < _NCH)
                def _prefetch():
                    pltpu.async_copy(gg_hbm.at[idx_v.at[c + 2]], buf, sem)
            return carry

        lax.fori_loop(0, _NCH // 2, pair_body, 0)
        pltpu.sync_copy(acc_v.at[0], st_hbm.at[wid])
        pltpu.sync_copy(acc_v.at[1], st_hbm.at[_NWORK + wid])

    return k(gg, idx3, xw1f)


def _merge_worker_stats(stv, nrow):
    s = q = None
    for p in range(SPLIT):
        o = p * 2 * nrow
        ps = jnp.sum(stv[o:o + nrow], axis=0, keepdims=True)
        pq = jnp.sum(stv[o + nrow:o + 2 * nrow], axis=0, keepdims=True)
        s = ps if s is None else s + ps
        q = pq if q is None else q + pq
    return s, q


def _mlp_body(y1_ref, st_ref, g_ref, be_ref, wa_ref, wb_ref, b11_ref,
              mx_ref, mn_ref, st2_ref):
    s, q = _merge_worker_stats(st_ref[...], _NWORK)
    mu = s * (1.0 / CNT1)
    var = q * (1.0 / CNT1) - mu * mu
    a = lax.rsqrt(var + 1e-5) * g_ref[...]
    bb = be_ref[...] - mu * a
    w = y1_ref[...]
    ya = lax.bitcast_convert_type(w << 16, jnp.float32)
    yb = lax.bitcast_convert_type(w & (-65536), jnp.float32)
    za = jnp.maximum(ya * a[:, :C1 // 2] + bb[:, :C1 // 2], 0.0)
    zb = jnp.maximum(yb * a[:, C1 // 2:] + bb[:, C1 // 2:], 0.0)
    y2 = jnp.dot(za, wa_ref[...], preferred_element_type=jnp.float32)
    y2 += jnp.dot(zb, wb_ref[...], preferred_element_type=jnp.float32)
    y2 += b11_ref[...]

    @pl.when((pl.program_id(0) == 0) & (pl.program_id(1) == 0))
    def _init():
        st2_ref[...] = jnp.zeros_like(st2_ref)

    st2_ref[0:1, :] += jnp.sum(y2, axis=0, keepdims=True)
    st2_ref[1:2, :] += jnp.sum(y2 * y2, axis=0, keepdims=True)
    y2r = y2.reshape(TM, NS, C2)
    mx_ref[0] = jnp.max(y2r, axis=1)
    mn_ref[0] = jnp.min(y2r, axis=1)


def _mlp(y1, st1cat, g1_0r, be1_0r, w11ta, w11tb, b11r):
    return pl.pallas_call(
        _mlp_body,
        grid=(BH, N // TM),
        in_specs=[
            pl.BlockSpec((TM * NS, C1 // 2),
                         lambda b, i: (b * (N // TM) + i, 0)),
            pl.BlockSpec((SPLIT * 2 * _NWORK, C1), lambda b, i: (0, 0)),
            pl.BlockSpec((1, C1), lambda b, i: (0, 0)),
            pl.BlockSpec((1, C1), lambda b, i: (0, 0)),
            pl.BlockSpec((C1 // 2, C2), lambda b, i: (0, 0)),
            pl.BlockSpec((C1 // 2, C2), lambda b, i: (0, 0)),
            pl.BlockSpec((1, C2), lambda b, i: (0, 0)),
        ],
        out_specs=[
            pl.BlockSpec((1, TM, C2), lambda b, i: (b, i, 0)),
            pl.BlockSpec((1, TM, C2), lambda b, i: (b, i, 0)),
            pl.BlockSpec((2, C2), lambda b, i: (0, 0)),
        ],
        out_shape=[
            jax.ShapeDtypeStruct((BH, N, C2), jnp.float32),
            jax.ShapeDtypeStruct((BH, N, C2), jnp.float32),
            jax.ShapeDtypeStruct((2, C2), jnp.float32),
        ],
    )(y1, st1cat, g1_0r, be1_0r, w11ta, w11tb, b11r)


def _head_body(mx_ref, mn_ref, feat1_ref, st_ref, g_ref, be_ref,
               w2at_ref, w2bt_ref, b2_ref, y3_ref, st3_ref):
    s, q = _merge_worker_stats(st_ref[...], 1)
    mu = s * (1.0 / CNT1)
    var = q * (1.0 / CNT1) - mu * mu
    g = g_ref[...]
    a = lax.rsqrt(var + 1e-5) * g
    bb = be_ref[...] - mu * a
    sel = jnp.where(g >= 0.0, mx_ref[0], mn_ref[0])
    h = jnp.maximum(sel * a + bb, 0.0)
    y3 = jnp.dot(h, w2at_ref[...], preferred_element_type=jnp.float32)
    y3 += jnp.dot(feat1_ref[0], w2bt_ref[...], preferred_element_type=jnp.float32)
    y3 += b2_ref[...]
    y3_ref[0] = y3

    @pl.when((pl.program_id(0) == 0) & (pl.program_id(1) == 0))
    def _init():
        st3_ref[...] = jnp.zeros_like(st3_ref)

    st3_ref[0:1, :] += jnp.sum(y3, axis=0, keepdims=True)
    st3_ref[1:2, :] += jnp.sum(y3 * y3, axis=0, keepdims=True)


def _head(mx, mn, feat1, st2cat, g1_1r, be1_1r, w2at, w2bt, b2r):
    return pl.pallas_call(
        _head_body,
        grid=(BH, N // TB),
        in_specs=[
            pl.BlockSpec((1, TB, C2), lambda b, i: (b, i, 0)),
            pl.BlockSpec((1, TB, C2), lambda b, i: (b, i, 0)),
            pl.BlockSpec((1, TB, D1), lambda b, i: (b, i, 0)),
            pl.BlockSpec((SPLIT * 2, C2), lambda b, i: (0, 0)),
            pl.BlockSpec((1, C2), lambda b, i: (0, 0)),
            pl.BlockSpec((1, C2), lambda b, i: (0, 0)),
            pl.BlockSpec((C2, C3), lambda b, i: (0, 0)),
            pl.BlockSpec((D1, C3), lambda b, i: (0, 0)),
            pl.BlockSpec((1, C3), lambda b, i: (0, 0)),
        ],
        out_specs=[
            pl.BlockSpec((1, TB, C3), lambda b, i: (b, i, 0)),
            pl.BlockSpec((2, C3), lambda b, i: (0, 0)),
        ],
        out_shape=[
            jax.ShapeDtypeStruct((BH, N, C3), jnp.float32),
            jax.ShapeDtypeStruct((2, C3), jnp.float32),
        ],
    )(mx, mn, feat1, st2cat, g1_1r, be1_1r, w2at, w2bt, b2r)


def _final_body(y3_ref, st_ref, g_ref, be_ref, out_ref):
    s, q = _merge_worker_stats(st_ref[...], 1)
    mu = s * (1.0 / CNT3)
    var = q * (1.0 / CNT3) - mu * mu
    a = lax.rsqrt(var + 1e-5) * g_ref[...]
    bb = be_ref[...] - mu * a
    out_ref[0] = jnp.maximum(y3_ref[0] * a + bb, 0.0)


def _final(y3, st3cat, g2r, be2r):
    return pl.pallas_call(
        _final_body,
        grid=(BH, N // TB),
        in_specs=[
            pl.BlockSpec((1, TB, C3), lambda b, i: (b, i, 0)),
            pl.BlockSpec((SPLIT * 2, C3), lambda b, i: (0, 0)),
            pl.BlockSpec((1, C3), lambda b, i: (0, 0)),
            pl.BlockSpec((1, C3), lambda b, i: (0, 0)),
        ],
        out_specs=pl.BlockSpec((1, TB, C3), lambda b, i: (b, i, 0)),
        out_shape=jax.ShapeDtypeStruct((BH, N, C3), jnp.float32),
    )(y3, st3cat, g2r, be2r)


def kernel(xyz1, xyz2, feat1, feat2, W1_0, b1_0, g1_0, be1_0,
           W1_1, b1_1, g1_1, be1_1, W2_0, b2_0, g2_0, be2_0):
    wft = jnp.transpose(W1_0[:, :D2])
    wxt = jnp.transpose(W1_0[:, D2:])
    w11t = jnp.transpose(W1_1)
    w2at = jnp.transpose(W2_0[:, :C2])
    w2bt = jnp.transpose(W2_0[:, C2:])
    r = lambda v: v.reshape(1, -1)

    y1p, st1p = [], []
    for h in range(SPLIT):
        sl = slice(h * BH, (h + 1) * BH)
        gg = _make_table(feat2[sl], xyz2[sl], wft, wxt, r(b1_0))
        idx, xw1 = _knn(xyz1[sl], jnp.transpose(xyz2[sl], (0, 2, 1)), wxt)
        idx3 = idx.reshape(_NWORK, _NCH, _CHUNK)
        y1, st1 = _sc_gather(gg, idx3, xw1)
        y1p.append(y1)
        st1p.append(st1)

    st1cat = jnp.concatenate(st1p, axis=0)[:, CACB]
    g1_0s, be1_0s = g1_0[CACB], be1_0[CACB]
    w11ta, w11tb = w11t[CA, :], w11t[CB, :]
    mxp, mnp, st2p = [], [], []
    for h in range(SPLIT):
        mx, mn, st = _mlp(y1p[h], st1cat, r(g1_0s), r(be1_0s),
                          w11ta, w11tb, r(b1_1))
        mxp.append(mx); mnp.append(mn); st2p.append(st)
    st2cat = jnp.concatenate(st2p, axis=0)
    y3p, st3p = [], []
    for h in range(SPLIT):
        sl = slice(h * BH, (h + 1) * BH)
        y3, st = _head(mxp[h], mnp[h], feat1[sl], st2cat,
                       r(g1_1), r(be1_1), w2at, w2bt, r(b2_0))
        y3p.append(y3); st3p.append(st)
    st3cat = jnp.concatenate(st3p, axis=0)
    outs = [_final(y3p[h], st3cat, r(g2_0), r(be2_0)) for h in range(SPLIT)]
    return jnp.concatenate(outs, axis=0)


# speedup vs baseline: 1.0399x; 1.0399x over previous
import functools

import numpy as np

import jax
import jax.numpy as jnp
from jax import lax
from jax.experimental import pallas as pl
from jax.experimental.pallas import tpu as pltpu
from jax.experimental.pallas import tpu_sc as plsc

B, N, S, NS = 8, 4096, 1024, 8
D1, D2 = 128, 256
C1 = 128
C2 = 64
C3 = 64
SPLIT = 4
BH = B // SPLIT
TN = 512
TM = 512
TB = 1024
CNT1 = float(B * N * NS)
CNT3 = float(B * N)
NV = C1 // 16

_j = np.arange(C1 // 2)
CA = (32 * (_j // 16) + (_j % 16)).astype(np.int32)
CB = CA + 16
CACB = np.concatenate([CA, CB])


def _table_body(feat2_ref, xyz2_ref, wft_ref, wxt_ref, b_ref, gg_ref):
    gg = jnp.dot(feat2_ref[0], wft_ref[...], preferred_element_type=jnp.float32)
    gg += jnp.dot(xyz2_ref[0], wxt_ref[...], preferred_element_type=jnp.float32)
    gg_ref[...] = gg + b_ref[...]


def _make_table(feat2, xyz2, wft, wxt, b1_0r):
    return pl.pallas_call(
        _table_body,
        grid=(BH,),
        in_specs=[
            pl.BlockSpec((1, S, D2), lambda b: (b, 0, 0)),
            pl.BlockSpec((1, S, 3), lambda b: (b, 0, 0)),
            pl.BlockSpec((D2, C1), lambda b: (0, 0)),
            pl.BlockSpec((3, C1), lambda b: (0, 0)),
            pl.BlockSpec((1, C1), lambda b: (0, 0)),
        ],
        out_specs=pl.BlockSpec((S, C1), lambda b: (b, 0)),
        out_shape=jax.ShapeDtypeStruct((BH * S, C1), jnp.float32),
    )(feat2, xyz2, wft, wxt, b1_0r)


def _knn_body(xyz1_ref, xyz2t_ref, wxt_ref, idx_ref, xw1_ref):
    b = pl.program_id(0)
    x1 = xyz1_ref[0]
    x2t = xyz2t_ref[0]
    d = -2.0 * jnp.dot(x1, x2t, preferred_element_type=jnp.float32)
    d += jnp.sum(x1 * x1, axis=1, keepdims=True)
    d += jnp.sum(x2t * x2t, axis=0, keepdims=True)
    iota = lax.broadcasted_iota(jnp.int32, (TN, S), 1)
    off = (b * S).astype(jnp.int32)
    for k in range(NS):
        idxk = jnp.argmin(d, axis=1).astype(jnp.int32)
        d = jnp.where(iota == idxk[:, None], jnp.inf, d)
        idx_ref[0, :, k] = idxk + off
    xw1_ref[...] = jnp.dot(x1, wxt_ref[...], preferred_element_type=jnp.float32)


def _knn(xyz1, xyz2t, wxt):
    return pl.pallas_call(
        _knn_body,
        grid=(BH, N // TN),
        in_specs=[
            pl.BlockSpec((1, TN, 3), lambda b, i: (b, i, 0)),
            pl.BlockSpec((1, 3, S), lambda b, i: (b, 0, 0)),
            pl.BlockSpec((3, C1), lambda b, i: (0, 0)),
        ],
        out_specs=[
            pl.BlockSpec((1, TN, NS), lambda b, i: (b, i, 0)),
            pl.BlockSpec((TN, C1), lambda b, i: (b * (N // TN) + i, 0)),
        ],
        out_shape=[
            jax.ShapeDtypeStruct((BH, N, NS), jnp.int32),
            jax.ShapeDtypeStruct((BH * N, C1), jnp.float32),
        ],
    )(xyz1, xyz2t, wxt)


_NROWS = BH * N * NS
_CHUNK = 128
_NWORK = 32
_RPW = _NROWS // _NWORK
_NCH = _RPW // _CHUNK
_PPW = _RPW // NS
_PPC = _CHUNK // NS


def _sc_gather(gg, idx3, xw1f):
    mesh = plsc.VectorSubcoreMesh(core_axis_name="c", subcore_axis_name="s")

    @functools.partial(
        pl.kernel,
        mesh=mesh,
        out_type=[
            jax.ShapeDtypeStruct((_NROWS, C1), jnp.float32),
            jax.ShapeDtypeStruct((2 * _NWORK, C1), jnp.float32),
        ],
        scratch_types=[
            pltpu.VMEM((_NCH, _CHUNK), jnp.int32),
            pltpu.VMEM((_PPW, C1), jnp.float32),
            pltpu.VMEM((_CHUNK, C1), jnp.float32),
            pltpu.VMEM((_CHUNK, C1), jnp.float32),
            pltpu.VMEM((2, C1), jnp.float32),
            pltpu.SemaphoreType.DMA,
            pltpu.SemaphoreType.DMA,
        ],
    )
    def k(gg_hbm, idx_hbm, xw_hbm, out_hbm, st_hbm,
          idx_v, xw_v, rows0, rows1, acc_v, sem0, sem1):
        wid = lax.axis_index("s") * 2 + lax.axis_index("c")
        base = wid * _RPW
        pltpu.sync_copy(idx_hbm.at[wid], idx_v)
        pltpu.sync_copy(xw_hbm.at[pl.ds(wid * _PPW, _PPW)], xw_v)
        z = jnp.zeros((16,), jnp.float32)
        for v in range(NV):
            acc_v[0, pl.ds(v * 16, 16)] = z
            acc_v[1, pl.ds(v * 16, 16)] = z
        bufs = (rows0, rows1)
        sems = (sem0, sem1)
        pltpu.async_copy(gg_hbm.at[idx_v.at[0]], rows0, sem0)
        pltpu.async_copy(gg_hbm.at[idx_v.at[1]], rows1, sem1)

        def pair_body(c2, carry):
            for par in range(2):
                c = c2 * 2 + par
                buf = bufs[par]
                sem = sems[par]
                pltpu.make_async_copy(
                    gg_hbm.at[pl.ds(0, _CHUNK)], buf, sem).wait()

                def point_body(p, carry2):
                    prow = p * NS
                    for v2 in range(NV // 2):
                        la = pl.ds(v2 * 32, 16)
                        lb = pl.ds(v2 * 32 + 16, 16)
                        lo = pl.ds(v2 * 16, 16)
                        xa = xw_v[c * _PPC + p, la]
                        xb = xw_v[c * _PPC + p, lb]
                        sa = acc_v[0, la]
                        qa = acc_v[1, la]
                        sb = acc_v[0, lb]
                        qb = acc_v[1, lb]
                        for rr in range(NS):
                            ya = buf[prow + rr, la] - xa
                            yb = buf[prow + rr, lb] - xb
                            buf[prow + rr, la] = ya
                            buf[prow + rr, lb] = yb
                            sa = sa + ya
                            qa = qa + ya * ya
                            sb = sb + yb
                            qb = qb + yb * yb
                        acc_v[0, la] = sa
                        acc_v[1, la] = qa
                        acc_v[0, lb] = sb
                        acc_v[1, lb] = qb
                    return carry2

                lax.fori_loop(0, _PPC, point_body, 0)
                pltpu.sync_copy(
                    buf, out_hbm.at[pl.ds(base + c * _CHUNK, _CHUNK)])

                @pl.when(c + 2 < _NCH)
                def _prefetch():
                    pltpu.async_copy(gg_hbm.at[idx_v.at[c + 2]], buf, sem)
            return carry

        lax.fori_loop(0, _NCH // 2, pair_body, 0)
        pltpu.sync_copy(acc_v.at[0], st_hbm.at[wid])
        pltpu.sync_copy(acc_v.at[1], st_hbm.at[_NWORK + wid])

    return k(gg, idx3, xw1f)


def _merge_worker_stats(stv, nrow):
    s = q = None
    for p in range(SPLIT):
        o = p * 2 * nrow
        ps = jnp.sum(stv[o:o + nrow], axis=0, keepdims=True)
        pq = jnp.sum(stv[o + nrow:o + 2 * nrow], axis=0, keepdims=True)
        s = ps if s is None else s + ps
        q = pq if q is None else q + pq
    return s, q


def _mlp_body(y1_ref, st_ref, g_ref, be_ref, w11t_ref, b11_ref,
              mx_ref, mn_ref, st2_ref):
    s, q = _merge_worker_stats(st_ref[...], _NWORK)
    mu = s * (1.0 / CNT1)
    var = q * (1.0 / CNT1) - mu * mu
    a = lax.rsqrt(var + 1e-5) * g_ref[...]
    bb = be_ref[...] - mu * a
    z = jnp.maximum(y1_ref[...] * a + bb, 0.0)
    y2 = jnp.dot(z, w11t_ref[...], preferred_element_type=jnp.float32)
    y2 += b11_ref[...]

    @pl.when((pl.program_id(0) == 0) & (pl.program_id(1) == 0))
    def _init():
        st2_ref[...] = jnp.zeros_like(st2_ref)

    st2_ref[0:1, :] += jnp.sum(y2, axis=0, keepdims=True)
    st2_ref[1:2, :] += jnp.sum(y2 * y2, axis=0, keepdims=True)
    y2r = y2.reshape(TM, NS, C2)
    mx_ref[0] = jnp.max(y2r, axis=1)
    mn_ref[0] = jnp.min(y2r, axis=1)


def _mlp(y1, st1cat, g1_0r, be1_0r, w11t, b11r):
    return pl.pallas_call(
        _mlp_body,
        grid=(BH, N // TM),
        in_specs=[
            pl.BlockSpec((TM * NS, C1),
                         lambda b, i: (b * (N // TM) + i, 0)),
            pl.BlockSpec((SPLIT * 2 * _NWORK, C1), lambda b, i: (0, 0)),
            pl.BlockSpec((1, C1), lambda b, i: (0, 0)),
            pl.BlockSpec((1, C1), lambda b, i: (0, 0)),
            pl.BlockSpec((C1, C2), lambda b, i: (0, 0)),
            pl.BlockSpec((1, C2), lambda b, i: (0, 0)),
        ],
        out_specs=[
            pl.BlockSpec((1, TM, C2), lambda b, i: (b, i, 0)),
            pl.BlockSpec((1, TM, C2), lambda b, i: (b, i, 0)),
            pl.BlockSpec((2, C2), lambda b, i: (0, 0)),
        ],
        out_shape=[
            jax.ShapeDtypeStruct((BH, N, C2), jnp.float32),
            jax.ShapeDtypeStruct((BH, N, C2), jnp.float32),
            jax.ShapeDtypeStruct((2, C2), jnp.float32),
        ],
    )(y1, st1cat, g1_0r, be1_0r, w11t, b11r)


def _head_body(mx_ref, mn_ref, feat1_ref, st_ref, g_ref, be_ref,
               w2at_ref, w2bt_ref, b2_ref, y3_ref, st3_ref):
    s, q = _merge_worker_stats(st_ref[...], 1)
    mu = s * (1.0 / CNT1)
    var = q * (1.0 / CNT1) - mu * mu
    g = g_ref[...]
    a = lax.rsqrt(var + 1e-5) * g
    bb = be_ref[...] - mu * a
    sel = jnp.where(g >= 0.0, mx_ref[0], mn_ref[0])
    h = jnp.maximum(sel * a + bb, 0.0)
    y3 = jnp.dot(h, w2at_ref[...], preferred_element_type=jnp.float32)
    y3 += jnp.dot(feat1_ref[0], w2bt_ref[...], preferred_element_type=jnp.float32)
    y3 += b2_ref[...]
    y3_ref[0] = y3

    @pl.when((pl.program_id(0) == 0) & (pl.program_id(1) == 0))
    def _init():
        st3_ref[...] = jnp.zeros_like(st3_ref)

    st3_ref[0:1, :] += jnp.sum(y3, axis=0, keepdims=True)
    st3_ref[1:2, :] += jnp.sum(y3 * y3, axis=0, keepdims=True)


def _head(mx, mn, feat1, st2cat, g1_1r, be1_1r, w2at, w2bt, b2r):
    return pl.pallas_call(
        _head_body,
        grid=(BH, N // TB),
        in_specs=[
            pl.BlockSpec((1, TB, C2), lambda b, i: (b, i, 0)),
            pl.BlockSpec((1, TB, C2), lambda b, i: (b, i, 0)),
            pl.BlockSpec((1, TB, D1), lambda b, i: (b, i, 0)),
            pl.BlockSpec((SPLIT * 2, C2), lambda b, i: (0, 0)),
            pl.BlockSpec((1, C2), lambda b, i: (0, 0)),
            pl.BlockSpec((1, C2), lambda b, i: (0, 0)),
            pl.BlockSpec((C2, C3), lambda b, i: (0, 0)),
            pl.BlockSpec((D1, C3), lambda b, i: (0, 0)),
            pl.BlockSpec((1, C3), lambda b, i: (0, 0)),
        ],
        out_specs=[
            pl.BlockSpec((1, TB, C3), lambda b, i: (b, i, 0)),
            pl.BlockSpec((2, C3), lambda b, i: (0, 0)),
        ],
        out_shape=[
            jax.ShapeDtypeStruct((BH, N, C3), jnp.float32),
            jax.ShapeDtypeStruct((2, C3), jnp.float32),
        ],
    )(mx, mn, feat1, st2cat, g1_1r, be1_1r, w2at, w2bt, b2r)


def _final_body(y3_ref, st_ref, g_ref, be_ref, out_ref):
    s, q = _merge_worker_stats(st_ref[...], 1)
    mu = s * (1.0 / CNT3)
    var = q * (1.0 / CNT3) - mu * mu
    a = lax.rsqrt(var + 1e-5) * g_ref[...]
    bb = be_ref[...] - mu * a
    out_ref[0] = jnp.maximum(y3_ref[0] * a + bb, 0.0)


def _final(y3, st3cat, g2r, be2r):
    return pl.pallas_call(
        _final_body,
        grid=(BH, N // TB),
        in_specs=[
            pl.BlockSpec((1, TB, C3), lambda b, i: (b, i, 0)),
            pl.BlockSpec((SPLIT * 2, C3), lambda b, i: (0, 0)),
            pl.BlockSpec((1, C3), lambda b, i: (0, 0)),
            pl.BlockSpec((1, C3), lambda b, i: (0, 0)),
        ],
        out_specs=pl.BlockSpec((1, TB, C3), lambda b, i: (b, i, 0)),
        out_shape=jax.ShapeDtypeStruct((BH, N, C3), jnp.float32),
    )(y3, st3cat, g2r, be2r)


def kernel(xyz1, xyz2, feat1, feat2, W1_0, b1_0, g1_0, be1_0,
           W1_1, b1_1, g1_1, be1_1, W2_0, b2_0, g2_0, be2_0):
    wft = jnp.transpose(W1_0[:, :D2])
    wxt = jnp.transpose(W1_0[:, D2:])
    w11t = jnp.transpose(W1_1)
    w2at = jnp.transpose(W2_0[:, :C2])
    w2bt = jnp.transpose(W2_0[:, C2:])
    r = lambda v: v.reshape(1, -1)

    y1p, st1p = [], []
    for h in range(SPLIT):
        sl = slice(h * BH, (h + 1) * BH)
        gg = _make_table(feat2[sl], xyz2[sl], wft, wxt, r(b1_0))
        idx, xw1 = _knn(xyz1[sl], jnp.transpose(xyz2[sl], (0, 2, 1)), wxt)
        idx3 = idx.reshape(_NWORK, _NCH, _CHUNK)
        y1, st1 = _sc_gather(gg, idx3, xw1)
        y1p.append(y1)
        st1p.append(st1)

    st1cat = jnp.concatenate(st1p, axis=0)
    mxp, mnp, st2p = [], [], []
    for h in range(SPLIT):
        mx, mn, st = _mlp(y1p[h], st1cat, r(g1_0), r(be1_0), w11t, r(b1_1))
        mxp.append(mx); mnp.append(mn); st2p.append(st)
    st2cat = jnp.concatenate(st2p, axis=0)
    y3p, st3p = [], []
    for h in range(SPLIT):
        sl = slice(h * BH, (h + 1) * BH)
        y3, st = _head(mxp[h], mnp[h], feat1[sl], st2cat,
                       r(g1_1), r(be1_1), w2at, w2bt, r(b2_0))
        y3p.append(y3); st3p.append(st)
    st3cat = jnp.concatenate(st3p, axis=0)
    outs = [_final(y3p[h], st3cat, r(g2_0), r(be2_0)) for h in range(SPLIT)]
    return jnp.concatenate(outs, axis=0)


# speedup vs baseline: 1.0413x; 1.0013x over previous
import functools

import jax
import jax.numpy as jnp
from jax import lax
from jax.experimental import pallas as pl
from jax.experimental.pallas import tpu as pltpu
from jax.experimental.pallas import tpu_sc as plsc

B, N, S, NS = 8, 4096, 1024, 8
D1, D2 = 128, 256
C1 = 128
C2 = 64
C3 = 64
SPLIT = 4
BH = B // SPLIT
TN = 512
TM = 512
TB = 1024
CNT1 = float(B * N * NS)
CNT3 = float(B * N)
NV = C1 // 16


def _table_body(feat2_ref, xyz2_ref, wft_ref, wxt_ref, b_ref, gg_ref):
    gg = jnp.dot(feat2_ref[0], wft_ref[...], preferred_element_type=jnp.float32)
    gg += jnp.dot(xyz2_ref[0], wxt_ref[...], preferred_element_type=jnp.float32)
    gg_ref[...] = gg + b_ref[...]


def _make_table(feat2, xyz2, wft, wxt, b1_0r):
    return pl.pallas_call(
        _table_body,
        grid=(BH,),
        in_specs=[
            pl.BlockSpec((1, S, D2), lambda b: (b, 0, 0)),
            pl.BlockSpec((1, S, 3), lambda b: (b, 0, 0)),
            pl.BlockSpec((D2, C1), lambda b: (0, 0)),
            pl.BlockSpec((3, C1), lambda b: (0, 0)),
            pl.BlockSpec((1, C1), lambda b: (0, 0)),
        ],
        out_specs=pl.BlockSpec((S, C1), lambda b: (b, 0)),
        out_shape=jax.ShapeDtypeStruct((BH * S, C1), jnp.float32),
    )(feat2, xyz2, wft, wxt, b1_0r)


def _knn_body(xyz1_ref, xyz2t_ref, wxt_ref, idx_ref, xw1_ref):
    b = pl.program_id(0)
    x1 = xyz1_ref[0]
    x2t = xyz2t_ref[0]
    d = -2.0 * jnp.dot(x1, x2t, preferred_element_type=jnp.float32)
    d += jnp.sum(x1 * x1, axis=1, keepdims=True)
    d += jnp.sum(x2t * x2t, axis=0, keepdims=True)
    iota = lax.broadcasted_iota(jnp.int32, (TN, S), 1)
    off = (b * S).astype(jnp.int32)
    for k in range(NS):
        idxk = jnp.argmin(d, axis=1).astype(jnp.int32)
        d = jnp.where(iota == idxk[:, None], jnp.inf, d)
        idx_ref[0, :, k] = idxk + off
    xw1_ref[...] = jnp.dot(x1, wxt_ref[...], preferred_element_type=jnp.float32)


def _knn(xyz1, xyz2t, wxt):
    return pl.pallas_call(
        _knn_body,
        grid=(BH, N // TN),
        in_specs=[
            pl.BlockSpec((1, TN, 3), lambda b, i: (b, i, 0)),
            pl.BlockSpec((1, 3, S), lambda b, i: (b, 0, 0)),
            pl.BlockSpec((3, C1), lambda b, i: (0, 0)),
        ],
        out_specs=[
            pl.BlockSpec((1, TN, NS), lambda b, i: (b, i, 0)),
            pl.BlockSpec((TN, C1), lambda b, i: (b * (N // TN) + i, 0)),
        ],
        out_shape=[
            jax.ShapeDtypeStruct((BH, N, NS), jnp.int32),
            jax.ShapeDtypeStruct((BH * N, C1), jnp.float32),
        ],
    )(xyz1, xyz2t, wxt)


_NROWS = BH * N * NS
_CHUNK = 128
_NWORK = 32
_RPW = _NROWS // _NWORK
_NCH = _RPW // _CHUNK
_PPW = _RPW // NS
_PPC = _CHUNK // NS


def _sc_gather(gg, idx3, xw1f):
    mesh = plsc.VectorSubcoreMesh(core_axis_name="c", subcore_axis_name="s")

    @functools.partial(
        pl.kernel,
        mesh=mesh,
        out_type=[
            jax.ShapeDtypeStruct((_NROWS, C1), jnp.float32),
            jax.ShapeDtypeStruct((2 * _NWORK, C1), jnp.float32),
        ],
        scratch_types=[
            pltpu.VMEM((_NCH, _CHUNK), jnp.int32),
            pltpu.VMEM((_PPW, C1), jnp.float32),
            pltpu.VMEM((_CHUNK, C1), jnp.float32),
            pltpu.VMEM((_CHUNK, C1), jnp.float32),
            pltpu.VMEM((2, C1), jnp.float32),
            pltpu.SemaphoreType.DMA,
            pltpu.SemaphoreType.DMA,
        ],
    )
    def k(gg_hbm, idx_hbm, xw_hbm, out_hbm, st_hbm,
          idx_v, xw_v, rows0, rows1, acc_v, sem0, sem1):
        wid = lax.axis_index("s") * 2 + lax.axis_index("c")
        base = wid * _RPW
        pltpu.sync_copy(idx_hbm.at[wid], idx_v)
        pltpu.sync_copy(xw_hbm.at[pl.ds(wid * _PPW, _PPW)], xw_v)
        z = jnp.zeros((16,), jnp.float32)
        for v in range(NV):
            acc_v[0, pl.ds(v * 16, 16)] = z
            acc_v[1, pl.ds(v * 16, 16)] = z
        bufs = (rows0, rows1)
        sems = (sem0, sem1)
        pltpu.async_copy(gg_hbm.at[idx_v.at[0]], rows0, sem0)
        pltpu.async_copy(gg_hbm.at[idx_v.at[1]], rows1, sem1)

        def pair_body(c2, carry):
            for par in range(2):
                c = c2 * 2 + par
                buf = bufs[par]
                sem = sems[par]
                pltpu.make_async_copy(
                    gg_hbm.at[pl.ds(0, _CHUNK)], buf, sem).wait()

                def point_body(p, carry2):
                    prow = p * NS
                    for v2 in range(NV // 2):
                        la = pl.ds(v2 * 32, 16)
                        lb = pl.ds(v2 * 32 + 16, 16)
                        xa = xw_v[c * _PPC + p, la]
                        xb = xw_v[c * _PPC + p, lb]
                        sa = acc_v[0, la]
                        qa = acc_v[1, la]
                        sb = acc_v[0, lb]
                        qb = acc_v[1, lb]
                        for rr in range(NS):
                            ya = buf[prow + rr, la] - xa
                            yb = buf[prow + rr, lb] - xb
                            buf[prow + rr, la] = ya
                            buf[prow + rr, lb] = yb
                            sa = sa + ya
                            qa = qa + ya * ya
                            sb = sb + yb
                            qb = qb + yb * yb
                        acc_v[0, la] = sa
                        acc_v[1, la] = qa
                        acc_v[0, lb] = sb
                        acc_v[1, lb] = qb
                    return carry2

                lax.fori_loop(0, _PPC, point_body, 0)
                pltpu.sync_copy(
                    buf, out_hbm.at[pl.ds(base + c * _CHUNK, _CHUNK)])

                @pl.when(c + 2 < _NCH)
                def _prefetch():
                    pltpu.async_copy(gg_hbm.at[idx_v.at[c + 2]], buf, sem)
            return carry

        lax.fori_loop(0, _NCH // 2, pair_body, 0)
        pltpu.sync_copy(acc_v.at[0], st_hbm.at[wid])
        pltpu.sync_copy(acc_v.at[1], st_hbm.at[_NWORK + wid])

    return k(gg, idx3, xw1f)


def _merge_worker_stats(stv, nrow):
    s = q = None
    for p in range(SPLIT):
        o = p * 2 * nrow
        ps = jnp.sum(stv[o:o + nrow], axis=0, keepdims=True)
        pq = jnp.sum(stv[o + nrow:o + 2 * nrow], axis=0, keepdims=True)
        s = ps if s is None else s + ps
        q = pq if q is None else q + pq
    return s, q


def _mlp_body(y1_ref, st_ref, g_ref, be_ref, w11t_ref, b11_ref,
              mx_ref, mn_ref, st2_ref):
    s, q = _merge_worker_stats(st_ref[...], _NWORK)
    mu = s * (1.0 / CNT1)
    var = q * (1.0 / CNT1) - mu * mu
    a = lax.rsqrt(var + 1e-5) * g_ref[...]
    bb = be_ref[...] - mu * a
    z = jnp.maximum(y1_ref[...] * a + bb, 0.0)
    y2 = jnp.dot(z, w11t_ref[...], preferred_element_type=jnp.float32)
    y2 += b11_ref[...]

    @pl.when((pl.program_id(0) == 0) & (pl.program_id(1) == 0))
    def _init():
        st2_ref[...] = jnp.zeros_like(st2_ref)

    st2_ref[0:1, :] += jnp.sum(y2, axis=0, keepdims=True)
    st2_ref[1:2, :] += jnp.sum(y2 * y2, axis=0, keepdims=True)
    y2r = y2.reshape(TM, NS, C2)
    mx_ref[0] = jnp.max(y2r, axis=1)
    mn_ref[0] = jnp.min(y2r, axis=1)


def _mlp(y1, st1cat, g1_0r, be1_0r, w11t, b11r):
    return pl.pallas_call(
        _mlp_body,
        grid=(BH, N // TM),
        in_specs=[
            pl.BlockSpec((TM * NS, C1),
                         lambda b, i: (b * (N // TM) + i, 0)),
            pl.BlockSpec((SPLIT * 2 * _NWORK, C1), lambda b, i: (0, 0)),
            pl.BlockSpec((1, C1), lambda b, i: (0, 0)),
            pl.BlockSpec((1, C1), lambda b, i: (0, 0)),
            pl.BlockSpec((C1, C2), lambda b, i: (0, 0)),
            pl.BlockSpec((1, C2), lambda b, i: (0, 0)),
        ],
        out_specs=[
            pl.BlockSpec((1, TM, C2), lambda b, i: (b, i, 0)),
            pl.BlockSpec((1, TM, C2), lambda b, i: (b, i, 0)),
            pl.BlockSpec((2, C2), lambda b, i: (0, 0)),
        ],
        out_shape=[
            jax.ShapeDtypeStruct((BH, N, C2), jnp.float32),
            jax.ShapeDtypeStruct((BH, N, C2), jnp.float32),
            jax.ShapeDtypeStruct((2, C2), jnp.float32),
        ],
    )(y1, st1cat, g1_0r, be1_0r, w11t, b11r)


def _head_body(mx_ref, mn_ref, feat1_ref, st_ref, g_ref, be_ref,
               w2at_ref, w2bt_ref, b2_ref, y3_ref, st3_ref):
    s, q = _merge_worker_stats(st_ref[...], 1)
    mu = s * (1.0 / CNT1)
    var = q * (1.0 / CNT1) - mu * mu
    g = g_ref[...]
    a = lax.rsqrt(var + 1e-5) * g
    bb = be_ref[...] - mu * a
    sel = jnp.where(g >= 0.0, mx_ref[0], mn_ref[0])
    h = jnp.maximum(sel * a + bb, 0.0)
    y3 = jnp.dot(h, w2at_ref[...], preferred_element_type=jnp.float32)
    y3 += jnp.dot(feat1_ref[0], w2bt_ref[...], preferred_element_type=jnp.float32)
    y3 += b2_ref[...]
    y3_ref[0] = y3

    @pl.when((pl.program_id(0) == 0) & (pl.program_id(1) == 0))
    def _init():
        st3_ref[...] = jnp.zeros_like(st3_ref)

    st3_ref[0:1, :] += jnp.sum(y3, axis=0, keepdims=True)
    st3_ref[1:2, :] += jnp.sum(y3 * y3, axis=0, keepdims=True)


def _head(mx, mn, feat1, st2cat, g1_1r, be1_1r, w2at, w2bt, b2r):
    return pl.pallas_call(
        _head_body,
        grid=(BH, N // TB),
        in_specs=[
            pl.BlockSpec((1, TB, C2), lambda b, i: (b, i, 0)),
            pl.BlockSpec((1, TB, C2), lambda b, i: (b, i, 0)),
            pl.BlockSpec((1, TB, D1), lambda b, i: (b, i, 0)),
            pl.BlockSpec((SPLIT * 2, C2), lambda b, i: (0, 0)),
            pl.BlockSpec((1, C2), lambda b, i: (0, 0)),
            pl.BlockSpec((1, C2), lambda b, i: (0, 0)),
            pl.BlockSpec((C2, C3), lambda b, i: (0, 0)),
            pl.BlockSpec((D1, C3), lambda b, i: (0, 0)),
            pl.BlockSpec((1, C3), lambda b, i: (0, 0)),
        ],
        out_specs=[
            pl.BlockSpec((1, TB, C3), lambda b, i: (b, i, 0)),
            pl.BlockSpec((2, C3), lambda b, i: (0, 0)),
        ],
        out_shape=[
            jax.ShapeDtypeStruct((BH, N, C3), jnp.float32),
            jax.ShapeDtypeStruct((2, C3), jnp.float32),
        ],
    )(mx, mn, feat1, st2cat, g1_1r, be1_1r, w2at, w2bt, b2r)


def _final_body(y3_ref, st_ref, g_ref, be_ref, out_ref):
    s, q = _merge_worker_stats(st_ref[...], 1)
    mu = s * (1.0 / CNT3)
    var = q * (1.0 / CNT3) - mu * mu
    a = lax.rsqrt(var + 1e-5) * g_ref[...]
    bb = be_ref[...] - mu * a
    out_ref[0] = jnp.maximum(y3_ref[0] * a + bb, 0.0)


def _final(y3, st3cat, g2r, be2r):
    return pl.pallas_call(
        _final_body,
        grid=(BH, N // TB),
        in_specs=[
            pl.BlockSpec((1, TB, C3), lambda b, i: (b, i, 0)),
            pl.BlockSpec((SPLIT * 2, C3), lambda b, i: (0, 0)),
            pl.BlockSpec((1, C3), lambda b, i: (0, 0)),
            pl.BlockSpec((1, C3), lambda b, i: (0, 0)),
        ],
        out_specs=pl.BlockSpec((1, TB, C3), lambda b, i: (b, i, 0)),
        out_shape=jax.ShapeDtypeStruct((BH, N, C3), jnp.float32),
    )(y3, st3cat, g2r, be2r)


def kernel(xyz1, xyz2, feat1, feat2, W1_0, b1_0, g1_0, be1_0,
           W1_1, b1_1, g1_1, be1_1, W2_0, b2_0, g2_0, be2_0):
    wft = jnp.transpose(W1_0[:, :D2])
    wxt = jnp.transpose(W1_0[:, D2:])
    w11t = jnp.transpose(W1_1)
    w2at = jnp.transpose(W2_0[:, :C2])
    w2bt = jnp.transpose(W2_0[:, C2:])
    r = lambda v: v.reshape(1, -1)

    y1p, st1p = [], []
    for h in range(SPLIT):
        sl = slice(h * BH, (h + 1) * BH)
        gg = _make_table(feat2[sl], xyz2[sl], wft, wxt, r(b1_0))
        idx, xw1 = _knn(xyz1[sl], jnp.transpose(xyz2[sl], (0, 2, 1)), wxt)
        idx3 = idx.reshape(_NWORK, _NCH, _CHUNK)
        y1, st1 = _sc_gather(gg, idx3, xw1)
        y1p.append(y1)
        st1p.append(st1)

    st1cat = jnp.concatenate(st1p, axis=0)
    mxp, mnp, st2p = [], [], []
    for h in range(SPLIT):
        mx, mn, st = _mlp(y1p[h], st1cat, r(g1_0), r(be1_0), w11t, r(b1_1))
        mxp.append(mx); mnp.append(mn); st2p.append(st)
    st2cat = jnp.concatenate(st2p, axis=0)
    y3p, st3p = [], []
    for h in range(SPLIT):
        sl = slice(h * BH, (h + 1) * BH)
        y3, st = _head(mxp[h], mnp[h], feat1[sl], st2cat,
                       r(g1_1), r(be1_1), w2at, w2bt, r(b2_0))
        y3p.append(y3); st3p.append(st)
    st3cat = jnp.concatenate(st3p, axis=0)
    outs = [_final(y3p[h], st3cat, r(g2_0), r(be2_0)) for h in range(SPLIT)]
    return jnp.concatenate(outs, axis=0)
